# baseline (speedup 1.0000x reference)
import jax
import jax.numpy as jnp
from jax import lax
from jax.experimental import pallas as pl
from jax.experimental.pallas import tpu as pltpu

_S = 64.0
_ADJ = 64.0 * 0.4

_BB = 8


def _body(labels_ref, x_ref, o_ref):
    b = pl.program_id(0)
    c = x_ref.shape[1]
    cols = lax.broadcasted_iota(jnp.int32, (1, c), 1)
    for r in range(x_ref.shape[0]):
        lab = labels_ref[b * _BB + r]
        hit = cols == lab
        o_ref[r : r + 1, :] = x_ref[r : r + 1, :] * _S - jnp.where(hit, _ADJ, 0.0)


def kernel(logits, labels, embeddings):
    B, C = logits.shape
    labels = labels.astype(jnp.int32)
    return pl.pallas_call(
        _body,
        grid=(B // _BB,),
        in_specs=[
            pl.BlockSpec(memory_space=pltpu.SMEM),
            pl.BlockSpec((_BB, C), lambda b: (b, 0)),
        ],
        out_specs=pl.BlockSpec((_BB, C), lambda b: (b, 0)),
        out_shape=jax.ShapeDtypeStruct((B, C), jnp.float32),
    )(labels, logits)


# speedup vs baseline: 1.1810x; 1.1810x over previous
import jax
import jax.numpy as jnp
from jax import lax
from jax.experimental import pallas as pl
from jax.experimental.pallas import tpu as pltpu

_S = 64.0
_ADJ = 64.0 * 0.4

_BB = 8


def _body(labels_ref, x_ref, o_ref):
    c = x_ref.shape[1]
    labs = labels_ref[...]
    cols = lax.broadcasted_iota(jnp.int32, (_BB, c), 1)
    hit = cols == labs
    o_ref[...] = x_ref[...] * _S - jnp.where(hit, _ADJ, 0.0)


def kernel(logits, labels, embeddings):
    B, C = logits.shape
    labels2d = labels.astype(jnp.int32).reshape(B, 1)
    return pl.pallas_call(
        _body,
        grid=(B // _BB,),
        in_specs=[
            pl.BlockSpec((_BB, 1), lambda b: (b, 0)),
            pl.BlockSpec((_BB, C), lambda b: (b, 0)),
        ],
        out_specs=pl.BlockSpec((_BB, C), lambda b: (b, 0)),
        out_shape=jax.ShapeDtypeStruct((B, C), jnp.float32),
    )(labels2d, logits)


# speedup vs baseline: 1.2014x; 1.0172x over previous
import jax
import jax.numpy as jnp
from jax import lax
from jax.experimental import pallas as pl
from jax.experimental.pallas import tpu as pltpu

_S = 64.0
_ADJ = 64.0 * 0.4

_BB = 16


def _body(labels_ref, x_ref, o_ref):
    c = x_ref.shape[1]
    labs = labels_ref[...]
    cols = lax.broadcasted_iota(jnp.int32, (_BB, c), 1)
    hit = cols == labs
    o_ref[...] = x_ref[...] * _S - jnp.where(hit, _ADJ, 0.0)


def kernel(logits, labels, embeddings):
    B, C = logits.shape
    labels2d = labels.astype(jnp.int32).reshape(B, 1)
    return pl.pallas_call(
        _body,
        grid=(B // _BB,),
        in_specs=[
            pl.BlockSpec((_BB, 1), lambda b: (b, 0)),
            pl.BlockSpec((_BB, C), lambda b: (b, 0)),
        ],
        out_specs=pl.BlockSpec((_BB, C), lambda b: (b, 0)),
        out_shape=jax.ShapeDtypeStruct((B, C), jnp.float32),
    )(labels2d, logits)
